# baseline (speedup 1.0000x reference)
import functools

import jax
import jax.numpy as jnp
from jax import lax
from jax.experimental import pallas as pl
from jax.experimental.pallas import tpu as pltpu
from jax.experimental.pallas import tpu_sc as plsc

HID = 4096
H4 = 1024
RD = 256
MEMN = 500
WSP = 512
TOPK = 3
FUSION = 0.5

NK = 8
BK = HID // NK
NB = 8
BB = HID // NB

NC, NS, L = 2, 16, 16
NW = NC * NS
RPW = 1024 // NW
NCH = WSP // L

_PD = lax.Precision.DEFAULT
_NEG = float("-inf")


def _dotT(a, b, prec=_PD):
    return lax.dot_general(a, b, (((1,), (1,)), ((), ())),
                           precision=prec, preferred_element_type=jnp.float32)


def _tc_a(state_ref, w1_ref, b1_ref, w2_ref, b2r_ref, bank_ref, mv_ref,
          rs_ref, ws_ref, h_s):
    i = pl.program_id(0)

    @pl.when(i == 0)
    def _k0():
        h_s[...] = _dotT(state_ref[...], w1_ref[...])

    @pl.when(i > 0)
    def _kacc():
        h_s[...] += _dotT(state_ref[...], w1_ref[...])

    @pl.when(i == NK - 1)
    def _tail():
        h = jnp.maximum(h_s[...] + b1_ref[...].reshape(1, H4), 0.0)
        rs = _dotT(h, w2_ref[...]) + b2r_ref[...].reshape(1, RD)
        rs_ref[...] = rs
        nrm = jnp.sqrt(jnp.sum(rs * rs, axis=1, keepdims=True))
        ns = rs / jnp.maximum(nrm, 1e-12)
        bk = bank_ref[...]
        bnrm = jnp.sqrt(jnp.sum(bk * bk, axis=1, keepdims=True))
        nb = bk / jnp.maximum(bnrm, 1e-12)
        sims = _dotT(ns, nb)
        ws = sims * (mv_ref[...].reshape(1, MEMN) + 1e-8)
        ws_ref[...] = jnp.concatenate(
            [ws, jnp.full((1024, WSP - MEMN), _NEG, jnp.float32)], axis=1)


def _sc_retrieve(ws_hbm, bank_hbm, out_hbm, ws_v, idx_v, rows_v, avg_v, sem):
    wid = lax.axis_index("s") * NC + lax.axis_index("c")
    base = wid * RPW
    pltpu.sync_copy(ws_hbm.at[pl.ds(base, RPW)], ws_v)
    lane = lax.iota(jnp.int32, L)
    zero = jnp.zeros((L,), jnp.int32)

    def _bcast_red(x, op):
        for s in (8, 4, 2, 1):
            x = op(x, x.at[lane ^ s].get(mode="promise_in_bounds"))
        return x

    def row_body(r, carry):
        picks = list(carry)
        m1 = jnp.full((L,), _NEG, jnp.float32)
        m2 = jnp.full((L,), _NEG, jnp.float32)
        m3 = jnp.full((L,), _NEG, jnp.float32)
        i1 = zero
        i2 = zero
        i3 = zero
        for c in range(NCH):
            v = ws_v[r, pl.ds(c * L, L)]
            iv = lane + (c * L)
            gt1 = v > m1
            dv = jnp.where(gt1, m1, v)
            di = jnp.where(gt1, i1, iv)
            m1 = jnp.where(gt1, v, m1)
            i1 = jnp.where(gt1, iv, i1)
            gt2 = dv > m2
            dv2 = jnp.where(gt2, m2, dv)
            di2 = jnp.where(gt2, i2, di)
            m2 = jnp.where(gt2, dv, m2)
            i2 = jnp.where(gt2, di, i2)
            gt3 = dv2 > m3
            m3 = jnp.where(gt3, dv2, m3)
            i3 = jnp.where(gt3, di2, i3)
        selA = lane == r
        selB = lane == (r - L)
        for k in range(TOPK):
            mm = jnp.maximum(jnp.maximum(m1, m2), m3)
            gv = _bcast_red(mm, jnp.maximum)
            e1 = jnp.where(m1 == gv, i1, WSP)
            e2 = jnp.where(m2 == gv, i2, WSP)
            e3 = jnp.where(m3 == gv, i3, WSP)
            gi = _bcast_red(jnp.minimum(jnp.minimum(e1, e2), e3),
                            jnp.minimum)
            picks[2 * k] = jnp.where(selA, gi, picks[2 * k])
            picks[2 * k + 1] = jnp.where(selB, gi, picks[2 * k + 1])
            m1 = jnp.where(i1 == gi, _NEG, m1)
            m2 = jnp.where(i2 == gi, _NEG, m2)
            m3 = jnp.where(i3 == gi, _NEG, m3)
        return tuple(picks)

    picks = lax.fori_loop(0, RPW, row_body, (zero,) * (2 * TOPK))
    for k in range(TOPK):
        idx_v[pl.ds(2 * k * L, L)] = picks[2 * k]
        idx_v[pl.ds((2 * k + 1) * L, L)] = picks[2 * k + 1]
    pltpu.async_copy(bank_hbm.at[idx_v], rows_v, sem).wait()

    def avg_body(r, carry):
        for c in range(RD // L):
            s = pl.ds(c * L, L)
            avg_v[r, s] = (rows_v[r, s] + rows_v[RPW + r, s]
                           + rows_v[2 * RPW + r, s]) / 3.0
        return carry

    lax.fori_loop(0, RPW, avg_body, 0)
    pltpu.sync_copy(avg_v, out_hbm.at[pl.ds(base, RPW)])


_sc_call = functools.partial(
    pl.kernel,
    _sc_retrieve,
    out_type=jax.ShapeDtypeStruct((1024, RD), jnp.float32),
    mesh=plsc.VectorSubcoreMesh(core_axis_name="c", subcore_axis_name="s"),
    scratch_types=[
        pltpu.VMEM((RPW, WSP), jnp.float32),
        pltpu.VMEM((TOPK * RPW,), jnp.int32),
        pltpu.VMEM((TOPK * RPW, RD), jnp.float32),
        pltpu.VMEM((RPW, RD), jnp.float32),
        pltpu.SemaphoreType.DMA,
    ],
)


def _tc_b(rs_ref, avg_ref, bank_ref, mv_ref, chw1_ref, chb1_ref, chw2_ref,
          chb2_ref, dirw_ref, dirb_ref, ssw_ref, ssb_ref, vw_ref, vb_ref,
          tpw1_ref, tpb1_ref, tpw2_ref, tpb2_ref,
          lat_ref, np_ref, p0_ref, act_ref, lp_ref, val_ref, ent_ref,
          nbank_ref, nvals_ref, g_s):
    i = pl.program_id(0)

    @pl.when(i == 0)
    def _epilogue():
        rs = rs_ref[...]
        rs_f = (1.0 - FUSION) * rs + FUSION * avg_ref[...]
        c1 = jnp.maximum(_dotT(rs_f, chw1_ref[...])
                         + chb1_ref[...].reshape(1, 128), 0.0)
        logits = _dotT(c1, chw2_ref[...]) + chb2_ref[...].reshape(1, 2)
        mx = jnp.max(logits, axis=1, keepdims=True)
        e = jnp.exp(logits - mx)
        p = e / jnp.sum(e, axis=1, keepdims=True)
        p0 = p[:, 0:1]
        p1 = p[:, 1:2]
        act = (p1 > p0).astype(jnp.int32)
        p0_ref[...] = p0.T
        act_ref[...] = act.T
        lp_ref[...] = jnp.log(jnp.where(act > 0, p1, p0)).T
        ent_ref[...] = (-(p0 * jnp.log(p0 + 1e-8)
                          + p1 * jnp.log(p1 + 1e-8))).T
        d0 = _dotT(rs_f, dirw_ref[...]) + dirb_ref[...].reshape(1, RD)
        dnrm = jnp.sqrt(jnp.sum(d0 * d0, axis=1, keepdims=True))
        dn = d0 / jnp.maximum(dnrm, 1e-12)
        ssz = (jnp.sum(rs_f * ssw_ref[...], axis=1, keepdims=True)
               + ssb_ref[...].reshape(1, 1))
        ssz = 2.0 / (1.0 + jnp.exp(-ssz))
        val = (jnp.sum(rs_f * vw_ref[...], axis=1, keepdims=True)
               + vb_ref[...].reshape(1, 1))
        val_ref[...] = val.T
        npos = rs_f + ssz * dn
        np_ref[...] = npos
        pos_mean = jnp.sum(npos, axis=0, keepdims=True) / 1024.0
        val_mean = jnp.sum(val) / 1024.0
        r0 = lax.broadcasted_iota(jnp.int32, (MEMN, RD), 0) == 0
        nbank_ref[...] = jnp.where(r0, pos_mean, bank_ref[...])
        mv = mv_ref[...].reshape(1, MEMN)
        c0 = lax.broadcasted_iota(jnp.int32, (1, MEMN), 1) == 0
        nvals_ref[...] = jnp.where(c0, val_mean, mv)
        g_s[...] = jnp.maximum(
            _dotT(npos, tpw1_ref[...]) + tpb1_ref[...].reshape(1, H4), 0.0)

    @pl.when(i > 0)
    def _phase_b():
        lat_ref[...] = (_dotT(g_s[...], tpw2_ref[...])
                        + tpb2_ref[...].reshape(1, BB))


def kernel(state, step_num, sp_W1, sp_b1, sp_W2, sp_b2, tp_W1, tp_b1, tp_W2,
           tp_b2, ch_W1, ch_b1, ch_W2, ch_b2, dir_W, dir_b, ss_W, ss_b, v_W,
           v_b, memory_bank, memory_values):
    f32 = jnp.float32
    se = jnp.sin(jnp.asarray(step_num, f32) * 0.5)
    b2r = sp_b2 + 0.1 * se

    const = lambda shape: pl.BlockSpec(shape, lambda i: (0,) * len(shape))
    rs, ws = pl.pallas_call(
        _tc_a,
        grid=(NK,),
        in_specs=[
            pl.BlockSpec((1024, BK), lambda i: (0, i)),
            pl.BlockSpec((1024, BK), lambda i: (0, i)),
            const((H4,)),
            const((RD, H4)),
            const((RD,)),
            const((MEMN, RD)),
            const((MEMN,)),
        ],
        out_specs=[const((1024, RD)), const((1024, WSP))],
        out_shape=(jax.ShapeDtypeStruct((1024, RD), f32),
                   jax.ShapeDtypeStruct((1024, WSP), f32)),
        scratch_shapes=[pltpu.VMEM((1024, H4), f32)],
        compiler_params=pltpu.CompilerParams(
            dimension_semantics=("arbitrary",)),
    )(state, sp_W1, sp_b1, sp_W2, b2r, memory_bank, memory_values)

    avg = _sc_call()(ws, memory_bank)

    outs = (
        jax.ShapeDtypeStruct((1024, HID), f32),
        jax.ShapeDtypeStruct((1024, RD), f32),
        jax.ShapeDtypeStruct((1, 1024), f32),
        jax.ShapeDtypeStruct((1, 1024), jnp.int32),
        jax.ShapeDtypeStruct((1, 1024), f32),
        jax.ShapeDtypeStruct((1, 1024), f32),
        jax.ShapeDtypeStruct((1, 1024), f32),
        jax.ShapeDtypeStruct((MEMN, RD), f32),
        jax.ShapeDtypeStruct((1, MEMN), f32),
    )
    (lat, npos, p0, act, lp, val, ent, nbank, nvals) = pl.pallas_call(
        _tc_b,
        grid=(1 + NB,),
        in_specs=[
            const((1024, RD)),
            const((1024, RD)),
            const((MEMN, RD)),
            const((MEMN,)),
            const((128, RD)),
            const((128,)),
            const((2, 128)),
            const((2,)),
            const((RD, RD)),
            const((RD,)),
            const((1, RD)),
            const((1,)),
            const((1, RD)),
            const((1,)),
            const((H4, RD)),
            const((H4,)),
            pl.BlockSpec((BB, H4), lambda i: (jnp.maximum(i - 1, 0), 0)),
            pl.BlockSpec((BB,), lambda i: (jnp.maximum(i - 1, 0),)),
        ],
        out_specs=[
            pl.BlockSpec((1024, BB), lambda i: (0, jnp.maximum(i - 1, 0))),
            const((1024, RD)),
            const((1, 1024)),
            const((1, 1024)),
            const((1, 1024)),
            const((1, 1024)),
            const((1, 1024)),
            const((MEMN, RD)),
            const((1, MEMN)),
        ],
        out_shape=outs,
        scratch_shapes=[pltpu.VMEM((1024, H4), f32)],
        compiler_params=pltpu.CompilerParams(
            dimension_semantics=("arbitrary",)),
    )(rs, avg, memory_bank, memory_values, ch_W1, ch_b1, ch_W2, ch_b2,
      dir_W, dir_b, ss_W, ss_b, v_W, v_b, tp_W1, tp_b1, tp_W2, tp_b2)

    return (lat, npos, p0[0], act[0], lp[0], val[0],
            ent[0], nbank, nvals[0])
